# Initial kernel scaffold; baseline (speedup 1.0000x reference)
import jax
import jax.numpy as jnp
from jax.experimental import pallas as pl


def kernel(x, edge_index, Wl1, bl1, Wr1, Wl2, bl2, Wr2, W1, b1, W2, b2, W3, b3):
    raise NotImplementedError("write your pallas kernel here")


# speedup vs baseline: 4.4822x; 4.4822x over previous
import jax
import jax.numpy as jnp
from jax import lax
from jax.experimental import pallas as pl
from jax.experimental.pallas import tpu as pltpu
from jax.experimental.pallas import tpu_sc as plsc

N = 10000
E = 320000
D = 128
NC = 2
NS = 16
NW = NC * NS
EPW = E // NW
CHUNK = 80
ITERS = EPW // CHUNK
NA = 10240
RPS = NA // NS

_f32 = jnp.float32


def _sc_aggregate(feat, dst, zeros, src=None, ones=None):
  mesh = plsc.VectorSubcoreMesh(core_axis_name="c", subcore_axis_name="s")
  gather = src is not None

  scratch = [
      pltpu.VMEM_SHARED((NA, D), _f32),
      pltpu.VMEM((CHUNK,), jnp.int32),
      pltpu.VMEM((CHUNK, D), _f32),
  ]
  if gather:
    scratch.append(pltpu.VMEM((CHUNK,), jnp.int32))
    scratch.append(pltpu.SemaphoreType.DMA)

  def body(*refs):
    if gather:
      (feat_hbm, dst_hbm, zeros_hbm, src_hbm,
       agg_hbm, acc, dst_v, rows_v, src_v, sem) = refs
    else:
      (feat_hbm, dst_hbm, zeros_hbm, ones_hbm,
       agg_hbm, acc, dst_v, rows_v) = refs
    c = lax.axis_index("c")
    s = lax.axis_index("s")
    wid = c * NS + s
    r0 = s * RPS

    pltpu.sync_copy(zeros_hbm.at[pl.ds(r0, RPS)], acc.at[pl.ds(r0, RPS)])
    if not gather:
      pltpu.sync_copy(ones_hbm, rows_v)
    plsc.subcore_barrier()

    @pl.loop(0, ITERS)
    def _(i):
      e0 = wid * EPW + i * CHUNK
      pltpu.sync_copy(dst_hbm.at[pl.ds(e0, CHUNK)], dst_v)
      if gather:
        pltpu.sync_copy(src_hbm.at[pl.ds(e0, CHUNK)], src_v)
        pltpu.async_copy(feat_hbm.at[src_v], rows_v, sem).wait()
      pltpu.sync_copy(rows_v, acc.at[dst_v], add=True)

    plsc.subcore_barrier()

    o0 = c * NA + r0
    pltpu.sync_copy(acc.at[pl.ds(r0, RPS)], agg_hbm.at[pl.ds(o0, RPS)])

  ins = (feat, dst, zeros, src if gather else ones)
  fn = pl.kernel(body,
                 out_type=jax.ShapeDtypeStruct((NC * NA, D), _f32),
                 mesh=mesh, scratch_types=tuple(scratch))
  return fn(*ins)


def _dot_t(a, w):
  return lax.dot_general(a, w, (((1,), (1,)), ((), ())),
                         precision=lax.Precision.HIGHEST,
                         preferred_element_type=_f32)


BLK = 2000


def _tc1_body(agg0, agg1, deg0, deg1, x, wl, bl, wr, out):
  deg = jnp.maximum(deg0[:, 0:1] + deg1[:, 0:1], 1.0)
  agg = (agg0[...] + agg1[...]) / deg
  t = _dot_t(agg, wl[...]) + bl[...] + _dot_t(x[...], wr[...])
  out[...] = jnp.maximum(t, 0.0)


def _tc2_body(agg0, agg1, deg0, deg1, h, wl, bl, wr,
              w1, b1, w2, b2, w3, b3, qp_out, h2_out):
  deg = jnp.maximum(deg0[:, 0:1] + deg1[:, 0:1], 1.0)
  agg = (agg0[...] + agg1[...]) / deg
  h2 = jnp.maximum(_dot_t(agg, wl[...]) + bl[...] + _dot_t(h[...], wr[...]),
                   0.0)
  h2_out[...] = h2
  q = jnp.maximum(_dot_t(h2, w1[...]) + b1[...], 0.0)
  q = jnp.maximum(_dot_t(q, w2[...]) + b2[...], 0.0)
  qp_out[...] = _dot_t(q, w3[...]) + b3[...]


def _row_spec(cols):
  return pl.BlockSpec((BLK, cols), lambda i: (i, 0))


def _full_spec(shape):
  return pl.BlockSpec(shape, lambda i: tuple(0 for _ in shape))


def kernel(x, edge_index, Wl1, bl1, Wr1, Wl2, bl2, Wr2,
           W1, b1, W2, b2, W3, b3):
  edge_index = edge_index.astype(jnp.int32)
  src = edge_index[0]
  dst = edge_index[1]
  zeros = jnp.zeros((NA, D), _f32)
  ones = jnp.ones((CHUNK, D), _f32)

  deg = _sc_aggregate(x, dst, zeros, src=None, ones=ones)
  agg1 = _sc_aggregate(x, dst, zeros, src=src)
  a0, a1 = agg1[:N], agg1[NA:NA + N]
  d0, d1 = deg[:N], deg[NA:NA + N]

  grid = (N // BLK,)
  h = pl.pallas_call(
      _tc1_body,
      grid=grid,
      in_specs=[
          _row_spec(D), _row_spec(D), _row_spec(D), _row_spec(D),
          _row_spec(D),
          _full_spec((D, D)), _full_spec((1, D)), _full_spec((D, D)),
      ],
      out_specs=_row_spec(D),
      out_shape=jax.ShapeDtypeStruct((N, D), _f32),
  )(a0, a1, d0, d1, x, Wl1, bl1.reshape(1, D), Wr1)

  agg2 = _sc_aggregate(h, dst, zeros, src=src)
  a0, a1 = agg2[:N], agg2[NA:NA + N]

  qp, h2 = pl.pallas_call(
      _tc2_body,
      grid=grid,
      in_specs=[
          _row_spec(D), _row_spec(D), _row_spec(D), _row_spec(D),
          _row_spec(D),
          _full_spec((D, D)), _full_spec((1, D)), _full_spec((D, D)),
          _full_spec((128, D)), _full_spec((1, 128)),
          _full_spec((64, 128)), _full_spec((1, 64)),
          _full_spec((10, 64)), _full_spec((1, 10)),
      ],
      out_specs=[_row_spec(10), _row_spec(D)],
      out_shape=[jax.ShapeDtypeStruct((N, 10), _f32),
                 jax.ShapeDtypeStruct((N, D), _f32)],
  )(a0, a1, d0, d1, h, Wl2, bl2.reshape(1, D), Wr2,
    W1, b1.reshape(1, 128), W2, b2.reshape(1, 64), W3, b3.reshape(1, 10))

  return (qp, h2)


# speedup vs baseline: 6.3490x; 1.4165x over previous
import jax
import jax.numpy as jnp
from jax import lax
from jax.experimental import pallas as pl
from jax.experimental.pallas import tpu as pltpu
from jax.experimental.pallas import tpu_sc as plsc

N = 10000
E = 320000
D = 128
NC = 2
NS = 16
NW = NC * NS
EPW = E // NW
CHUNK = 80
ITERS = EPW // CHUNK
NA = 10240
RPS = NA // NS

_f32 = jnp.float32


NPAIR = (ITERS + 1) // 2


def _sc_aggregate(feat, dst, zeros, src=None, ones=None):
  mesh = plsc.VectorSubcoreMesh(core_axis_name="c", subcore_axis_name="s")
  gather = src is not None

  scratch = [
      pltpu.VMEM_SHARED((NA, D), _f32),
      pltpu.VMEM((CHUNK,), jnp.int32),
      pltpu.VMEM((CHUNK,), jnp.int32),
      pltpu.SemaphoreType.DMA,
      pltpu.SemaphoreType.DMA,
  ]
  if gather:
    scratch += [
        pltpu.VMEM((CHUNK,), jnp.int32),
        pltpu.VMEM((CHUNK,), jnp.int32),
        pltpu.VMEM((CHUNK, D), _f32),
        pltpu.VMEM((CHUNK, D), _f32),
        pltpu.SemaphoreType.DMA,
        pltpu.SemaphoreType.DMA,
    ]
  else:
    scratch.append(pltpu.VMEM((CHUNK, D), _f32))

  def body(*refs):
    if gather:
      (feat_hbm, dst_hbm, zeros_hbm, src_hbm, agg_hbm,
       acc, dstA, dstB, ssA, ssB, srcA, srcB, rowsA, rowsB,
       gsA, gsB) = refs
      dst_v = (dstA, dstB)
      src_v = (srcA, srcB)
      rows_v = (rowsA, rowsB)
      gsem = (gsA, gsB)
      ssem = (ssA, ssB)
    else:
      (feat_hbm, dst_hbm, zeros_hbm, ones_hbm, agg_hbm,
       acc, dstA, dstB, ssA, ssB, ones_v) = refs
      dst_v = (dstA, dstB)
      rows_v = (ones_v, ones_v)
      ssem = (ssA, ssB)
    c = lax.axis_index("c")
    s = lax.axis_index("s")
    wid = c * NS + s
    r0 = s * RPS

    def load_idx(x, ci):
      e0 = wid * EPW + ci * CHUNK
      pltpu.sync_copy(dst_hbm.at[pl.ds(e0, CHUNK)], dst_v[x])
      if gather:
        pltpu.sync_copy(src_hbm.at[pl.ds(e0, CHUNK)], src_v[x])
        pltpu.async_copy(feat_hbm.at[src_v[x]], rows_v[x], gsem[x])

    def wait_gather(x):
      pltpu.make_async_copy(feat_hbm.at[src_v[x]], rows_v[x],
                            gsem[x]).wait()

    def scatter(x):
      if gather:
        wait_gather(x)
      pltpu.async_copy(rows_v[x], acc.at[dst_v[x]], ssem[x], add=True)

    def wait_scatter(x):
      pltpu.make_async_copy(rows_v[x], acc.at[dst_v[x]], ssem[x]).wait()

    pltpu.sync_copy(zeros_hbm.at[pl.ds(r0, RPS)], acc.at[pl.ds(r0, RPS)])
    if not gather:
      pltpu.sync_copy(ones_hbm, ones_v)
    plsc.subcore_barrier()

    load_idx(0, 0)

    @pl.loop(0, NPAIR)
    def _(k):
      cB = 2 * k + 1
      cA2 = 2 * k + 2

      @pl.when(jnp.logical_and(cB < ITERS, k > 0))
      def _():
        wait_scatter(1)

      @pl.when(cB < ITERS)
      def _():
        load_idx(1, cB)

      scatter(0)

      @pl.when(cA2 < ITERS)
      def _():
        wait_scatter(0)
        load_idx(0, cA2)

      @pl.when(cB < ITERS)
      def _():
        scatter(1)

    wait_scatter(0)
    wait_scatter(1)
    plsc.subcore_barrier()

    o0 = c * NA + r0
    pltpu.sync_copy(acc.at[pl.ds(r0, RPS)], agg_hbm.at[pl.ds(o0, RPS)])

  ins = (feat, dst, zeros, src if gather else ones)
  fn = pl.kernel(body,
                 out_type=jax.ShapeDtypeStruct((NC * NA, D), _f32),
                 mesh=mesh, scratch_types=tuple(scratch))
  return fn(*ins)


def _dot_t(a, w):
  return lax.dot_general(a, w, (((1,), (1,)), ((), ())),
                         precision=lax.Precision.HIGHEST,
                         preferred_element_type=_f32)


BLK = 2000


def _tc1_body(agg0, agg1, deg0, deg1, x, wl, bl, wr, out):
  deg = jnp.maximum(deg0[:, 0:1] + deg1[:, 0:1], 1.0)
  agg = (agg0[...] + agg1[...]) / deg
  t = _dot_t(agg, wl[...]) + bl[...] + _dot_t(x[...], wr[...])
  out[...] = jnp.maximum(t, 0.0)


def _tc2_body(agg0, agg1, deg0, deg1, h, wl, bl, wr,
              w1, b1, w2, b2, w3, b3, qp_out, h2_out):
  deg = jnp.maximum(deg0[:, 0:1] + deg1[:, 0:1], 1.0)
  agg = (agg0[...] + agg1[...]) / deg
  h2 = jnp.maximum(_dot_t(agg, wl[...]) + bl[...] + _dot_t(h[...], wr[...]),
                   0.0)
  h2_out[...] = h2
  q = jnp.maximum(_dot_t(h2, w1[...]) + b1[...], 0.0)
  q = jnp.maximum(_dot_t(q, w2[...]) + b2[...], 0.0)
  qp_out[...] = _dot_t(q, w3[...]) + b3[...]


def _row_spec(cols):
  return pl.BlockSpec((BLK, cols), lambda i: (i, 0))


def _full_spec(shape):
  return pl.BlockSpec(shape, lambda i: tuple(0 for _ in shape))


def kernel(x, edge_index, Wl1, bl1, Wr1, Wl2, bl2, Wr2,
           W1, b1, W2, b2, W3, b3):
  edge_index = edge_index.astype(jnp.int32)
  src = edge_index[0]
  dst = edge_index[1]
  zeros = jnp.zeros((NA, D), _f32)
  ones = jnp.ones((CHUNK, D), _f32)

  deg = _sc_aggregate(x, dst, zeros, src=None, ones=ones)
  agg1 = _sc_aggregate(x, dst, zeros, src=src)
  a0, a1 = agg1[:N], agg1[NA:NA + N]
  d0, d1 = deg[:N], deg[NA:NA + N]

  grid = (N // BLK,)
  h = pl.pallas_call(
      _tc1_body,
      grid=grid,
      in_specs=[
          _row_spec(D), _row_spec(D), _row_spec(D), _row_spec(D),
          _row_spec(D),
          _full_spec((D, D)), _full_spec((1, D)), _full_spec((D, D)),
      ],
      out_specs=_row_spec(D),
      out_shape=jax.ShapeDtypeStruct((N, D), _f32),
  )(a0, a1, d0, d1, x, Wl1, bl1.reshape(1, D), Wr1)

  agg2 = _sc_aggregate(h, dst, zeros, src=src)
  a0, a1 = agg2[:N], agg2[NA:NA + N]

  qp, h2 = pl.pallas_call(
      _tc2_body,
      grid=grid,
      in_specs=[
          _row_spec(D), _row_spec(D), _row_spec(D), _row_spec(D),
          _row_spec(D),
          _full_spec((D, D)), _full_spec((1, D)), _full_spec((D, D)),
          _full_spec((128, D)), _full_spec((1, 128)),
          _full_spec((64, 128)), _full_spec((1, 64)),
          _full_spec((10, 64)), _full_spec((1, 10)),
      ],
      out_specs=[_row_spec(10), _row_spec(D)],
      out_shape=[jax.ShapeDtypeStruct((N, 10), _f32),
                 jax.ShapeDtypeStruct((N, D), _f32)],
  )(a0, a1, d0, d1, h, Wl2, bl2.reshape(1, D), Wr2,
    W1, b1.reshape(1, 128), W2, b2.reshape(1, 64), W3, b3.reshape(1, 10))

  return (qp, h2)


# speedup vs baseline: 7.2693x; 1.1449x over previous
import jax
import jax.numpy as jnp
from jax import lax
from jax.experimental import pallas as pl
from jax.experimental.pallas import tpu as pltpu
from jax.experimental.pallas import tpu_sc as plsc

N = 10000
E = 320000
D = 128
NC = 2
NS = 16
NW = NC * NS
EPW = E // NW
CHUNK = 80
ITERS = EPW // CHUNK
NPAIR = (ITERS + 1) // 2
NA = 10240
RPS = NA // NS
NHI = NA // 128

_f32 = jnp.float32


def _sc_aggregate(feat, dst, zeros, src):
  mesh = plsc.VectorSubcoreMesh(core_axis_name="c", subcore_axis_name="s")

  scratch = [
      pltpu.VMEM_SHARED((NA, D), _f32),
      pltpu.VMEM((CHUNK,), jnp.int32),
      pltpu.VMEM((CHUNK,), jnp.int32),
      pltpu.VMEM((CHUNK,), jnp.int32),
      pltpu.VMEM((CHUNK,), jnp.int32),
      pltpu.VMEM((CHUNK, D), _f32),
      pltpu.VMEM((CHUNK, D), _f32),
      pltpu.SemaphoreType.DMA,
      pltpu.SemaphoreType.DMA,
      pltpu.SemaphoreType.DMA,
      pltpu.SemaphoreType.DMA,
  ]

  def body(feat_hbm, dst_hbm, zeros_hbm, src_hbm, p0_hbm, p1_hbm,
           acc, dstA, dstB, srcA, srcB, rowsA, rowsB,
           ssA, ssB, gsA, gsB):
    dst_v = (dstA, dstB)
    src_v = (srcA, srcB)
    rows_v = (rowsA, rowsB)
    gsem = (gsA, gsB)
    ssem = (ssA, ssB)
    c = lax.axis_index("c")
    s = lax.axis_index("s")
    wid = c * NS + s
    r0 = s * RPS

    def load_idx(x, ci):
      e0 = wid * EPW + ci * CHUNK
      pltpu.sync_copy(dst_hbm.at[pl.ds(e0, CHUNK)], dst_v[x])
      pltpu.sync_copy(src_hbm.at[pl.ds(e0, CHUNK)], src_v[x])
      pltpu.async_copy(feat_hbm.at[src_v[x]], rows_v[x], gsem[x])

    def scatter(x):
      pltpu.make_async_copy(feat_hbm.at[src_v[x]], rows_v[x],
                            gsem[x]).wait()
      pltpu.async_copy(rows_v[x], acc.at[dst_v[x]], ssem[x], add=True)

    def wait_scatter(x):
      pltpu.make_async_copy(rows_v[x], acc.at[dst_v[x]], ssem[x]).wait()

    pltpu.sync_copy(zeros_hbm.at[pl.ds(r0, RPS)], acc.at[pl.ds(r0, RPS)])
    plsc.subcore_barrier()

    load_idx(0, 0)

    @pl.loop(0, NPAIR)
    def _(k):
      cB = 2 * k + 1
      cA2 = 2 * k + 2

      @pl.when(jnp.logical_and(cB < ITERS, k > 0))
      def _():
        wait_scatter(1)

      @pl.when(cB < ITERS)
      def _():
        load_idx(1, cB)

      scatter(0)

      @pl.when(cA2 < ITERS)
      def _():
        wait_scatter(0)
        load_idx(0, cA2)

      @pl.when(cB < ITERS)
      def _():
        scatter(1)

    wait_scatter(0)
    wait_scatter(1)
    plsc.subcore_barrier()

    @pl.when(c == 0)
    def _():
      pltpu.sync_copy(acc.at[pl.ds(r0, RPS)], p0_hbm.at[pl.ds(r0, RPS)])

    @pl.when(c == 1)
    def _():
      pltpu.sync_copy(acc.at[pl.ds(r0, RPS)], p1_hbm.at[pl.ds(r0, RPS)])

  fn = pl.kernel(body,
                 out_type=(jax.ShapeDtypeStruct((NA, D), _f32),
                           jax.ShapeDtypeStruct((NA, D), _f32)),
                 mesh=mesh, scratch_types=tuple(scratch))
  return fn(feat, dst, zeros, src)


EB = 8000


def _deg_body(dst_blk, out):
  i = pl.program_id(0)
  d = dst_blk[...]
  hi = d // 128
  lo = d % 128
  ihi = lax.broadcasted_iota(jnp.int32, (1, NHI), 1)
  ilo = lax.broadcasted_iota(jnp.int32, (1, 128), 1)
  oh_hi = (hi == ihi).astype(jnp.bfloat16)
  oh_lo = (lo == ilo).astype(jnp.bfloat16)
  part = lax.dot_general(oh_hi, oh_lo, (((0,), (0,)), ((), ())),
                         preferred_element_type=_f32)

  @pl.when(i == 0)
  def _():
    out[...] = jnp.zeros_like(out)

  out[...] += part


def _degrees(dst):
  deg2d = pl.pallas_call(
      _deg_body,
      grid=(E // EB,),
      in_specs=[pl.BlockSpec((EB, 1), lambda i: (i, 0))],
      out_specs=pl.BlockSpec((NHI, 128), lambda i: (0, 0)),
      out_shape=jax.ShapeDtypeStruct((NHI, 128), _f32),
  )(dst.reshape(E, 1))
  return deg2d.reshape(NA, 1)


def _dot_t(a, w):
  return lax.dot_general(a, w, (((1,), (1,)), ((), ())),
                         precision=lax.Precision.HIGHEST,
                         preferred_element_type=_f32)


BLK = 2000


def _tc1_body(agg0, agg1, deg, x, wl, bl, wr, out):
  d = jnp.maximum(deg[...], 1.0)
  agg = (agg0[...] + agg1[...]) / d
  t = _dot_t(agg, wl[...]) + bl[...] + _dot_t(x[...], wr[...])
  out[...] = jnp.maximum(t, 0.0)


def _tc2_body(agg0, agg1, deg, h, wl, bl, wr,
              w1, b1, w2, b2, w3, b3, qp_out, h2_out):
  d = jnp.maximum(deg[...], 1.0)
  agg = (agg0[...] + agg1[...]) / d
  h2 = jnp.maximum(_dot_t(agg, wl[...]) + bl[...] + _dot_t(h[...], wr[...]),
                   0.0)
  h2_out[...] = h2
  q = jnp.maximum(_dot_t(h2, w1[...]) + b1[...], 0.0)
  q = jnp.maximum(_dot_t(q, w2[...]) + b2[...], 0.0)
  qp_out[...] = _dot_t(q, w3[...]) + b3[...]


def _row_spec(cols):
  return pl.BlockSpec((BLK, cols), lambda i: (i, 0))


def _full_spec(shape):
  return pl.BlockSpec(shape, lambda i: tuple(0 for _ in shape))


def kernel(x, edge_index, Wl1, bl1, Wr1, Wl2, bl2, Wr2,
           W1, b1, W2, b2, W3, b3):
  edge_index = edge_index.astype(jnp.int32)
  src = edge_index[0]
  dst = edge_index[1]
  zeros = jnp.zeros((NA, D), _f32)

  deg = _degrees(dst)
  p0, p1 = _sc_aggregate(x, dst, zeros, src)

  grid = (N // BLK,)
  h = pl.pallas_call(
      _tc1_body,
      grid=grid,
      in_specs=[
          _row_spec(D), _row_spec(D), _row_spec(1), _row_spec(D),
          _full_spec((D, D)), _full_spec((1, D)), _full_spec((D, D)),
      ],
      out_specs=_row_spec(D),
      out_shape=jax.ShapeDtypeStruct((N, D), _f32),
  )(p0, p1, deg, x, Wl1, bl1.reshape(1, D), Wr1)

  p0, p1 = _sc_aggregate(h, dst, zeros, src)

  qp, h2 = pl.pallas_call(
      _tc2_body,
      grid=grid,
      in_specs=[
          _row_spec(D), _row_spec(D), _row_spec(1), _row_spec(D),
          _full_spec((D, D)), _full_spec((1, D)), _full_spec((D, D)),
          _full_spec((128, D)), _full_spec((1, 128)),
          _full_spec((64, 128)), _full_spec((1, 64)),
          _full_spec((10, 64)), _full_spec((1, 10)),
      ],
      out_specs=[_row_spec(10), _row_spec(D)],
      out_shape=[jax.ShapeDtypeStruct((N, 10), _f32),
                 jax.ShapeDtypeStruct((N, D), _f32)],
  )(p0, p1, deg, h, Wl2, bl2.reshape(1, D), Wr2,
    W1, b1.reshape(1, 128), W2, b2.reshape(1, 64), W3, b3.reshape(1, 10))

  return (qp, h2)


# speedup vs baseline: 8.0868x; 1.1125x over previous
import jax
import jax.numpy as jnp
from jax import lax
from jax.experimental import pallas as pl
from jax.experimental.pallas import tpu as pltpu
from jax.experimental.pallas import tpu_sc as plsc

N = 10000
E = 320000
D = 128
NC = 2
NS = 16
NW = NC * NS
CHUNK = 128
EP = NW * 10240
EPW = EP // NW
ITERS = EPW // CHUNK
CROWS = EP // CHUNK
CPS = ITERS
NHALF = 2
KG = ITERS // NHALF
NPAIR_H = KG // 2
NA = 10240
RPS = NA // NS
NHI = NA // 128

_f32 = jnp.float32


def _sc_aggregate(feat, dst2, zeros, src2):
  mesh = plsc.VectorSubcoreMesh(core_axis_name="c", subcore_axis_name="s")

  scratch = [
      pltpu.VMEM_SHARED((NA, D), _f32),
      pltpu.VMEM((KG, CHUNK), jnp.int32),
      pltpu.VMEM((KG, CHUNK), jnp.int32),
      pltpu.VMEM((CHUNK, D), _f32),
      pltpu.VMEM((CHUNK, D), _f32),
      pltpu.SemaphoreType.DMA,
      pltpu.SemaphoreType.DMA,
      pltpu.SemaphoreType.DMA,
      pltpu.SemaphoreType.DMA,
  ]

  def body(feat_hbm, dst_hbm, zeros_hbm, src_hbm, p0_hbm, p1_hbm,
           acc, dst_v, src_v, rowsA, rowsB, ssA, ssB, gsA, gsB):
    rows_v = (rowsA, rowsB)
    gsem = (gsA, gsB)
    ssem = (ssA, ssB)
    c = lax.axis_index("c")
    s = lax.axis_index("s")
    wid = c * NS + s
    r0 = s * RPS

    def gather(x, j):
      pltpu.async_copy(feat_hbm.at[src_v.at[j]], rows_v[x], gsem[x])

    def scatter(x, j):
      pltpu.make_async_copy(feat_hbm.at[src_v.at[j]], rows_v[x],
                            gsem[x]).wait()
      pltpu.async_copy(rows_v[x], acc.at[dst_v.at[j]], ssem[x], add=True)

    def wait_scatter(x, j):
      pltpu.make_async_copy(rows_v[x], acc.at[dst_v.at[j]],
                            ssem[x]).wait()

    pltpu.sync_copy(zeros_hbm.at[pl.ds(r0, RPS)], acc.at[pl.ds(r0, RPS)])
    plsc.subcore_barrier()

    @pl.loop(0, NHALF)
    def _(half):
      base = wid * CPS + half * KG
      pltpu.sync_copy(dst_hbm.at[pl.ds(base, KG)], dst_v)
      pltpu.sync_copy(src_hbm.at[pl.ds(base, KG)], src_v)

      gather(0, 0)

      @pl.loop(0, NPAIR_H)
      def _(k):
        cA = 2 * k
        cB = 2 * k + 1
        cA2 = 2 * k + 2

        @pl.when(k > 0)
        def _():
          wait_scatter(1, cB - 2)

        gather(1, cB)
        scatter(0, cA)

        @pl.when(k + 1 < NPAIR_H)
        def _():
          wait_scatter(0, cA)
          gather(0, cA2)

        scatter(1, cB)

      wait_scatter(0, KG - 2)
      wait_scatter(1, KG - 1)

    plsc.subcore_barrier()

    @pl.when(c == 0)
    def _():
      pltpu.sync_copy(acc.at[pl.ds(r0, RPS)], p0_hbm.at[pl.ds(r0, RPS)])

    @pl.when(c == 1)
    def _():
      pltpu.sync_copy(acc.at[pl.ds(r0, RPS)], p1_hbm.at[pl.ds(r0, RPS)])

  fn = pl.kernel(body,
                 out_type=(jax.ShapeDtypeStruct((NA, D), _f32),
                           jax.ShapeDtypeStruct((NA, D), _f32)),
                 mesh=mesh, scratch_types=tuple(scratch))
  return fn(feat, dst2, zeros, src2)


EB = 8000


def _deg_body(dst_blk, out):
  i = pl.program_id(0)
  d = dst_blk[...]
  hi = d // 128
  lo = d % 128
  ihi = lax.broadcasted_iota(jnp.int32, (1, NHI), 1)
  ilo = lax.broadcasted_iota(jnp.int32, (1, 128), 1)
  oh_hi = (hi == ihi).astype(jnp.bfloat16)
  oh_lo = (lo == ilo).astype(jnp.bfloat16)
  part = lax.dot_general(oh_hi, oh_lo, (((0,), (0,)), ((), ())),
                         preferred_element_type=_f32)

  @pl.when(i == 0)
  def _():
    out[...] = jnp.zeros_like(out)

  out[...] += part


def _degrees(dst):
  deg2d = pl.pallas_call(
      _deg_body,
      grid=(E // EB,),
      in_specs=[pl.BlockSpec((EB, 1), lambda i: (i, 0))],
      out_specs=pl.BlockSpec((NHI, 128), lambda i: (0, 0)),
      out_shape=jax.ShapeDtypeStruct((NHI, 128), _f32),
  )(dst.reshape(E, 1))
  return deg2d.reshape(NA, 1)


def _dot_t(a, w):
  return lax.dot_general(a, w, (((1,), (1,)), ((), ())),
                         precision=lax.Precision.HIGHEST,
                         preferred_element_type=_f32)


BLK = 2000


def _tc1_body(agg0, agg1, deg, x, wl, bl, wr, out):
  d = jnp.maximum(deg[...], 1.0)
  agg = (agg0[...] + agg1[...]) / d
  t = _dot_t(agg, wl[...]) + bl[...] + _dot_t(x[...], wr[...])
  out[...] = jnp.maximum(t, 0.0)


def _tc2_body(agg0, agg1, deg, h, wl, bl, wr,
              w1, b1, w2, b2, w3, b3, qp_out, h2_out):
  d = jnp.maximum(deg[...], 1.0)
  agg = (agg0[...] + agg1[...]) / d
  h2 = jnp.maximum(_dot_t(agg, wl[...]) + bl[...] + _dot_t(h[...], wr[...]),
                   0.0)
  h2_out[...] = h2
  q = jnp.maximum(_dot_t(h2, w1[...]) + b1[...], 0.0)
  q = jnp.maximum(_dot_t(q, w2[...]) + b2[...], 0.0)
  qp_out[...] = _dot_t(q, w3[...]) + b3[...]


def _row_spec(cols):
  return pl.BlockSpec((BLK, cols), lambda i: (i, 0))


def _full_spec(shape):
  return pl.BlockSpec(shape, lambda i: tuple(0 for _ in shape))


def kernel(x, edge_index, Wl1, bl1, Wr1, Wl2, bl2, Wr2,
           W1, b1, W2, b2, W3, b3):
  edge_index = edge_index.astype(jnp.int32)
  src = edge_index[0]
  dst = edge_index[1]
  zeros = jnp.zeros((NA, D), _f32)

  pad = EP - E
  pad_iota = jnp.arange(pad, dtype=jnp.int32)
  src2 = jnp.concatenate([src, pad_iota % N]).reshape(CROWS, CHUNK)
  dst2 = jnp.concatenate([dst, N + pad_iota % (NA - N)]).reshape(
      CROWS, CHUNK)

  deg = _degrees(dst)
  p0, p1 = _sc_aggregate(x, dst2, zeros, src2)

  grid = (N // BLK,)
  h = pl.pallas_call(
      _tc1_body,
      grid=grid,
      in_specs=[
          _row_spec(D), _row_spec(D), _row_spec(1), _row_spec(D),
          _full_spec((D, D)), _full_spec((1, D)), _full_spec((D, D)),
      ],
      out_specs=_row_spec(D),
      out_shape=jax.ShapeDtypeStruct((N, D), _f32),
  )(p0, p1, deg, x, Wl1, bl1.reshape(1, D), Wr1)

  p0, p1 = _sc_aggregate(h, dst2, zeros, src2)

  qp, h2 = pl.pallas_call(
      _tc2_body,
      grid=grid,
      in_specs=[
          _row_spec(D), _row_spec(D), _row_spec(1), _row_spec(D),
          _full_spec((D, D)), _full_spec((1, D)), _full_spec((D, D)),
          _full_spec((128, D)), _full_spec((1, 128)),
          _full_spec((64, 128)), _full_spec((1, 64)),
          _full_spec((10, 64)), _full_spec((1, 10)),
      ],
      out_specs=[_row_spec(10), _row_spec(D)],
      out_shape=[jax.ShapeDtypeStruct((N, 10), _f32),
                 jax.ShapeDtypeStruct((N, D), _f32)],
  )(p0, p1, deg, h, Wl2, bl2.reshape(1, D), Wr2,
    W1, b1.reshape(1, 128), W2, b2.reshape(1, 64), W3, b3.reshape(1, 10))

  return (qp, h2)
